# baseline (speedup 1.0000x reference)
import jax
import jax.numpy as jnp
from jax.experimental import pallas as pl

TOKENS = 16384
EMBED = 2048
NUM_EXPERTS = 64
ACTIVE_EXPERTS = 8

BLOCK_T = 2048
HALF = EMBED // 2

_NEG = -1e30


def _router_kernel(x1_ref, x2_ref, w_ref, b_ref, out_ref, idx_ref):
    w = w_ref[...]
    st = jax.lax.dot_general(
        w[:, :HALF], x1_ref[...], (((1,), (1,)), ((), ())),
        preferred_element_type=jnp.float32,
    )
    st = st + jax.lax.dot_general(
        w[:, HALF:], x2_ref[...], (((1,), (1,)), ((), ())),
        preferred_element_type=jnp.float32,
    )
    st = st + b_ref[...]

    iota = jax.lax.broadcasted_iota(jnp.int32, st.shape, 0)
    work = st
    chosen = jnp.zeros(st.shape, dtype=jnp.bool_)
    idx_rows = []
    for _ in range(ACTIVE_EXPERTS):
        m = jnp.max(work, axis=0, keepdims=True)
        idx = jnp.min(
            jnp.where(work == m, iota, NUM_EXPERTS), axis=0, keepdims=True
        )
        hit = iota == idx
        work = jnp.where(hit, _NEG, work)
        chosen = jnp.logical_or(chosen, hit)
        idx_rows.append(idx)

    mask = jnp.where(chosen, st, 0.0)
    mx = jnp.max(mask, axis=0, keepdims=True)
    e = jnp.exp(mask - mx)
    sm = e / jnp.sum(e, axis=0, keepdims=True)
    out_ref[...] = sm.T
    idx_ref[...] = jnp.concatenate(idx_rows, axis=0).T


@jax.jit
def kernel(inputs, W, b):
    b2 = b.reshape(NUM_EXPERTS, 1)
    grid = (TOKENS // BLOCK_T,)
    out, idx = pl.pallas_call(
        _router_kernel,
        grid=grid,
        in_specs=[
            pl.BlockSpec((BLOCK_T, HALF), lambda i: (i, 0)),
            pl.BlockSpec((BLOCK_T, HALF), lambda i: (i, 1)),
            pl.BlockSpec((NUM_EXPERTS, EMBED), lambda i: (0, 0)),
            pl.BlockSpec((NUM_EXPERTS, 1), lambda i: (0, 0)),
        ],
        out_specs=[
            pl.BlockSpec((BLOCK_T, NUM_EXPERTS), lambda i: (i, 0)),
            pl.BlockSpec((BLOCK_T, ACTIVE_EXPERTS), lambda i: (i, 0)),
        ],
        out_shape=[
            jax.ShapeDtypeStruct((TOKENS, NUM_EXPERTS), jnp.float32),
            jax.ShapeDtypeStruct((TOKENS, ACTIVE_EXPERTS), jnp.int32),
        ],
    )(inputs, inputs, W, b2)
    return (out, idx)
